# baseline (speedup 1.0000x reference)
import functools

import jax
import jax.numpy as jnp
from jax import lax
from jax.experimental import pallas as pl
from jax.experimental.pallas import tpu as pltpu
from jax.experimental.pallas import tpu_sc as plsc

BATCH = 4096
HIST_LEN = 200
EMBED_DIM = 64
B = BATCH * HIST_LEN
NC = 2
NS = 16
NW = NC * NS
BPW = B // NW
SUB = 128
CHUNK = 512
N_SUB = CHUNK // SUB
N_CHUNKS = BPW // CHUNK
IDX_ROWS = BPW // SUB


def _make_kernel():
    mesh = plsc.VectorSubcoreMesh(core_axis_name="c", subcore_axis_name="s")

    @functools.partial(
        pl.kernel,
        out_type=jax.ShapeDtypeStruct((B, EMBED_DIM), jnp.float32),
        mesh=mesh,
        scratch_types=[
            pltpu.VMEM((IDX_ROWS, SUB), jnp.int32),
            pltpu.VMEM((CHUNK, EMBED_DIM), jnp.float32),
            pltpu.SemaphoreType.DMA,
        ],
        compiler_params=pltpu.CompilerParams(use_tc_tiling_on_sc=False),
    )
    def gather_kernel(table_hbm, idx_hbm, out_hbm, idx_v, rows_v, gsem):
        wid = lax.axis_index("s") * NC + lax.axis_index("c")
        base = wid * BPW
        pltpu.sync_copy(idx_hbm.at[pl.ds(wid * IDX_ROWS, IDX_ROWS)], idx_v)

        def chunk_body(c, carry):
            cps = []
            for j in range(N_SUB):
                cp = pltpu.async_copy(
                    table_hbm.at[idx_v.at[c * N_SUB + j]],
                    rows_v.at[pl.ds(j * SUB, SUB)],
                    gsem,
                )
                cps.append(cp)
            for cp in cps:
                cp.wait()
            pltpu.sync_copy(rows_v, out_hbm.at[pl.ds(base + c * CHUNK, CHUNK)])
            return carry

        lax.fori_loop(0, N_CHUNKS, chunk_body, 0)

    return gather_kernel


_gather = _make_kernel()


@jax.jit
def kernel(questions, embedding_weights):
    idx = questions.reshape(B // SUB, SUB).astype(jnp.int32)
    out = _gather(embedding_weights, idx)
    return out.reshape(BATCH, HIST_LEN, EMBED_DIM)


# speedup vs baseline: 1.0218x; 1.0218x over previous
import functools

import jax
import jax.numpy as jnp
from jax import lax
from jax.experimental import pallas as pl
from jax.experimental.pallas import tpu as pltpu
from jax.experimental.pallas import tpu_sc as plsc

BATCH = 4096
HIST_LEN = 200
EMBED_DIM = 64
B = BATCH * HIST_LEN
NC = 2
NS = 16
NW = NC * NS
BPW = B // NW
SUB = 128
CHUNK = 512
N_SUB = CHUNK // SUB
N_CHUNKS = BPW // CHUNK
IDX_ROWS = BPW // SUB


def _make_kernel():
    mesh = plsc.VectorSubcoreMesh(core_axis_name="c", subcore_axis_name="s")

    @functools.partial(
        pl.kernel,
        out_type=jax.ShapeDtypeStruct((B, EMBED_DIM), jnp.float32),
        mesh=mesh,
        scratch_types=[
            pltpu.VMEM((IDX_ROWS, SUB), jnp.int32),
            pltpu.VMEM((2, CHUNK, EMBED_DIM), jnp.float32),
            pltpu.SemaphoreType.DMA,
            pltpu.SemaphoreType.DMA,
            pltpu.SemaphoreType.DMA,
        ],
        compiler_params=pltpu.CompilerParams(use_tc_tiling_on_sc=False),
    )
    def gather_kernel(table_hbm, idx_hbm, out_hbm, idx_v, rows_v, gsem, wsem0, wsem1):
        wid = lax.axis_index("s") * NC + lax.axis_index("c")
        base = wid * BPW
        pltpu.sync_copy(idx_hbm.at[pl.ds(wid * IDX_ROWS, IDX_ROWS)], idx_v)

        wsems = (wsem0, wsem1)

        def do_chunk(c, b, first):
            wb = pltpu.make_async_copy(
                rows_v.at[b],
                out_hbm.at[pl.ds(base + c * CHUNK, CHUNK)],
                wsems[b],
            )
            if not first:
                wb.wait()
            cps = []
            for j in range(N_SUB):
                cp = pltpu.async_copy(
                    table_hbm.at[idx_v.at[c * N_SUB + j]],
                    rows_v.at[b].at[pl.ds(j * SUB, SUB)],
                    gsem,
                )
                cps.append(cp)
            for cp in cps:
                cp.wait()
            wb.start()

        def pair_body(p, carry):
            for b in range(2):
                do_chunk(p * 2 + b, b, first=False)
            return carry

        for b in range(2):
            do_chunk(b, b, first=True)
        lax.fori_loop(1, N_CHUNKS // 2, pair_body, 0)
        for b in range(2):
            pltpu.make_async_copy(
                rows_v.at[b],
                out_hbm.at[pl.ds(base, CHUNK)],
                wsems[b],
            ).wait()

    return gather_kernel


_gather = _make_kernel()


@jax.jit
def kernel(questions, embedding_weights):
    idx = questions.reshape(B // SUB, SUB).astype(jnp.int32)
    out = _gather(embedding_weights, idx)
    return out.reshape(BATCH, HIST_LEN, EMBED_DIM)
